# Initial kernel scaffold; baseline (speedup 1.0000x reference)
import jax
import jax.numpy as jnp
from jax.experimental import pallas as pl


def kernel(x, adj, gcn_weight, cheb_weight, bias, alpha_cheb, gamma_param):
    raise NotImplementedError("write your pallas kernel here")


# speedup vs baseline: 1.7678x; 1.7678x over previous
import jax
import jax.numpy as jnp
from jax.experimental import pallas as pl
from jax.experimental.pallas import tpu as pltpu

N = 4096
D = 128
BR = 256
NB = N // BR
K1 = 3


def _body(x_ref, adj_ref, wg_ref, wc_ref, bias_ref, alpha_ref, gamma_ref,
          out_ref, abf, sbf, xsbf, t1bf, ts1bf, deg, wcat):
    p = pl.program_id(0)
    i = pl.program_id(1)
    rows = pl.ds(i * BR, BR)

    @pl.when(p == 0)
    def _phase0():
        @pl.when(i == 0)
        def _():
            s = jnp.dot(x_ref[...], wg_ref[...],
                        preferred_element_type=jnp.float32)
            sbf[...] = s.astype(jnp.bfloat16)

        a = adj_ref[...]
        ab = a.astype(jnp.bfloat16)
        abf[rows, :] = ab
        deg[rows, :] = jnp.sum(a, axis=1, keepdims=True)
        out_ref[rows, :] = jnp.dot(ab, sbf[...],
                                   preferred_element_type=jnp.float32)

    @pl.when(p == 1)
    def _phase1():
        @pl.when(i == 0)
        def _():
            dinv = jax.lax.rsqrt(jnp.maximum(deg[...], 1e-6))
            deg[...] = dinv
            xsbf[...] = (deg[...] * x_ref[...]).astype(jnp.bfloat16)

        dinv_r = deg[rows, :]
        mm = jnp.dot(abf[rows, :], xsbf[...],
                     preferred_element_type=jnp.float32)
        t1 = -dinv_r * mm
        t1bf[rows, :] = t1.astype(jnp.bfloat16)
        ts1bf[rows, :] = (dinv_r * t1).astype(jnp.bfloat16)

    @pl.when(p == 2)
    def _phase2():
        @pl.when(i == 0)
        def _():
            g = gamma_ref[0:1, 0:1]
            for k in range(K1):
                wk = wc_ref[k]
                fro = jnp.sqrt(jnp.sum(wk * wk))
                nwk = wk + g * fro
                wcat[pl.ds(k * D, D), :] = nwk.astype(jnp.bfloat16)

        xblk = x_ref[rows, :]
        dinv_r = deg[rows, :]
        m2 = jnp.dot(abf[rows, :], ts1bf[...],
                     preferred_element_type=jnp.float32)
        t2 = -2.0 * dinv_r * m2 - xblk
        basis = jnp.concatenate(
            [xblk.astype(jnp.bfloat16), t1bf[rows, :], t2.astype(jnp.bfloat16)],
            axis=1)
        oc = jnp.dot(basis, wcat[...],
                     preferred_element_type=jnp.float32) * 0.001
        aa = 1.0 / (1.0 + jnp.exp(-alpha_ref[0:1, 0:1]))
        out_ref[rows, :] = (aa * out_ref[rows, :] + (1.0 - aa) * oc
                            + bias_ref[...])


def kernel(x, adj, gcn_weight, cheb_weight, bias, alpha_cheb, gamma_param):
    bias2 = bias.reshape(1, D)
    alpha2 = alpha_cheb.reshape(1, 1)
    gamma2 = gamma_param.reshape(1, 1)

    return pl.pallas_call(
        _body,
        grid=(3, NB),
        in_specs=[
            pl.BlockSpec((N, D), lambda p, i: (0, 0)),
            pl.BlockSpec((BR, N),
                         lambda p, i: (jnp.where(p == 0, i, 0), 0)),
            pl.BlockSpec((D, D), lambda p, i: (0, 0)),
            pl.BlockSpec((K1, D, D), lambda p, i: (0, 0, 0)),
            pl.BlockSpec((1, D), lambda p, i: (0, 0)),
            pl.BlockSpec((1, 1), lambda p, i: (0, 0)),
            pl.BlockSpec((1, 1), lambda p, i: (0, 0)),
        ],
        out_specs=pl.BlockSpec((N, D), lambda p, i: (0, 0)),
        out_shape=jax.ShapeDtypeStruct((N, D), jnp.float32),
        scratch_shapes=[
            pltpu.VMEM((N, N), jnp.bfloat16),
            pltpu.VMEM((N, D), jnp.bfloat16),
            pltpu.VMEM((N, D), jnp.bfloat16),
            pltpu.VMEM((N, D), jnp.bfloat16),
            pltpu.VMEM((N, D), jnp.bfloat16),
            pltpu.VMEM((N, 1), jnp.float32),
            pltpu.VMEM((K1 * D, D), jnp.bfloat16),
        ],
        compiler_params=pltpu.CompilerParams(
            dimension_semantics=("arbitrary", "arbitrary"),
            vmem_limit_bytes=100 * 1024 * 1024,
        ),
    )(x, adj, gcn_weight, cheb_weight, bias2, alpha2, gamma2)


# speedup vs baseline: 1.9525x; 1.1045x over previous
import jax
import jax.numpy as jnp
from jax.experimental import pallas as pl
from jax.experimental.pallas import tpu as pltpu

N = 4096
D = 128
BR = 512
NB = N // BR
K1 = 3


def _body(x_ref, adj_ref, wg_ref, wc_ref, bias_ref, alpha_ref, gamma_ref,
          out_ref, abf, sbf, xsbf, t1bf, ts1bf, deg, wcat):
    p = pl.program_id(0)
    i = pl.program_id(1)
    rows = pl.ds(i * BR, BR)

    @pl.when(p == 0)
    def _phase0():
        @pl.when(i == 0)
        def _():
            s = jnp.dot(x_ref[...], wg_ref[...],
                        preferred_element_type=jnp.float32)
            sbf[...] = s.astype(jnp.bfloat16)

        a = adj_ref[...]
        ab = a.astype(jnp.bfloat16)
        abf[rows, :] = ab
        deg[rows, :] = jnp.sum(a, axis=1, keepdims=True)
        out_ref[rows, :] = jnp.dot(ab, sbf[...],
                                   preferred_element_type=jnp.float32)

    @pl.when(p == 1)
    def _phase1():
        @pl.when(i == 0)
        def _():
            dinv = jax.lax.rsqrt(jnp.maximum(deg[...], 1e-6))
            deg[...] = dinv
            xsbf[...] = (deg[...] * x_ref[...]).astype(jnp.bfloat16)

        dinv_r = deg[rows, :]
        mm = jnp.dot(abf[rows, :], xsbf[...],
                     preferred_element_type=jnp.float32)
        t1 = -dinv_r * mm
        t1bf[rows, :] = t1.astype(jnp.bfloat16)
        ts1bf[rows, :] = (dinv_r * t1).astype(jnp.bfloat16)

    @pl.when(p == 2)
    def _phase2():
        @pl.when(i == 0)
        def _():
            g = gamma_ref[0:1, 0:1]
            for k in range(K1):
                wk = wc_ref[k]
                fro = jnp.sqrt(jnp.sum(wk * wk))
                nwk = wk + g * fro
                wcat[pl.ds(k * D, D), :] = nwk.astype(jnp.bfloat16)

        xblk = x_ref[rows, :]
        dinv_r = deg[rows, :]
        m2 = jnp.dot(abf[rows, :], ts1bf[...],
                     preferred_element_type=jnp.float32)
        t2 = -2.0 * dinv_r * m2 - xblk
        basis = jnp.concatenate(
            [xblk.astype(jnp.bfloat16), t1bf[rows, :], t2.astype(jnp.bfloat16)],
            axis=1)
        oc = jnp.dot(basis, wcat[...],
                     preferred_element_type=jnp.float32) * 0.001
        aa = 1.0 / (1.0 + jnp.exp(-alpha_ref[0:1, 0:1]))
        out_ref[rows, :] = (aa * out_ref[rows, :] + (1.0 - aa) * oc
                            + bias_ref[...])


def kernel(x, adj, gcn_weight, cheb_weight, bias, alpha_cheb, gamma_param):
    bias2 = bias.reshape(1, D)
    alpha2 = alpha_cheb.reshape(1, 1)
    gamma2 = gamma_param.reshape(1, 1)

    return pl.pallas_call(
        _body,
        grid=(3, NB),
        in_specs=[
            pl.BlockSpec((N, D), lambda p, i: (0, 0)),
            pl.BlockSpec((BR, N),
                         lambda p, i: (jnp.where(p == 0, i, 0), 0)),
            pl.BlockSpec((D, D), lambda p, i: (0, 0)),
            pl.BlockSpec((K1, D, D), lambda p, i: (0, 0, 0)),
            pl.BlockSpec((1, D), lambda p, i: (0, 0)),
            pl.BlockSpec((1, 1), lambda p, i: (0, 0)),
            pl.BlockSpec((1, 1), lambda p, i: (0, 0)),
        ],
        out_specs=pl.BlockSpec((N, D), lambda p, i: (0, 0)),
        out_shape=jax.ShapeDtypeStruct((N, D), jnp.float32),
        scratch_shapes=[
            pltpu.VMEM((N, N), jnp.bfloat16),
            pltpu.VMEM((N, D), jnp.bfloat16),
            pltpu.VMEM((N, D), jnp.bfloat16),
            pltpu.VMEM((N, D), jnp.bfloat16),
            pltpu.VMEM((N, D), jnp.bfloat16),
            pltpu.VMEM((N, 1), jnp.float32),
            pltpu.VMEM((K1 * D, D), jnp.bfloat16),
        ],
        compiler_params=pltpu.CompilerParams(
            dimension_semantics=("arbitrary", "arbitrary"),
            vmem_limit_bytes=100 * 1024 * 1024,
        ),
    )(x, adj, gcn_weight, cheb_weight, bias2, alpha2, gamma2)
